# baseline (speedup 1.0000x reference)
import functools

import jax
import jax.numpy as jnp
from jax import lax
from jax.experimental import pallas as pl
from jax.experimental.pallas import tpu as pltpu
from jax.experimental.pallas import tpu_sc as plsc


def _make_sc_add(N):
    info = plsc.get_sparse_core_info()
    NC, NS = info.num_cores, info.num_subcores
    NW = NC * NS
    EPW = N // NW
    CE = 32768
    n_chunks = EPW // CE
    mesh = plsc.VectorSubcoreMesh(core_axis_name="c", subcore_axis_name="s")

    @functools.partial(
        pl.kernel,
        out_type=jax.ShapeDtypeStruct((N,), jnp.float32),
        mesh=mesh,
        scratch_types=[
            pltpu.VMEM((CE,), jnp.float32),
            pltpu.VMEM((CE,), jnp.float32),
        ],
    )
    def sc_add(x_hbm, e_hbm, out_hbm, bufx, bufe):
        wid = lax.axis_index("s") * NC + lax.axis_index("c")
        w_base = wid * EPW

        def chunk_body(k, carry):
            base = pl.multiple_of(w_base + k * CE, CE)
            pltpu.sync_copy(x_hbm.at[pl.ds(base, CE)], bufx)
            pltpu.sync_copy(e_hbm.at[pl.ds(base, CE)], bufe)

            @plsc.parallel_loop(0, CE, 16, unroll=8)
            def _body(i):
                plsc.addupdate(bufx.at[pl.ds(i, 16)], bufe[pl.ds(i, 16)])

            pltpu.sync_copy(bufx, out_hbm.at[pl.ds(base, CE)])
            return carry

        lax.fori_loop(0, n_chunks, chunk_body, 0)

    return sc_add


def kernel(x, node_pos_emb):
    B, L, D = x.shape
    N = B * L * D
    x1 = x.reshape(N)
    e1 = node_pos_emb.reshape(N)
    out = _make_sc_add(N)(x1, e1)
    return out.reshape(B, L, D)


# speedup vs baseline: 3.4073x; 3.4073x over previous
import functools

import jax
import jax.numpy as jnp
from jax import lax
from jax.experimental import pallas as pl
from jax.experimental.pallas import tpu as pltpu
from jax.experimental.pallas import tpu_sc as plsc


def _make_sc_add(R, D):
    info = plsc.get_sparse_core_info()
    NC, NS = info.num_cores, info.num_subcores
    NW = NC * NS
    CH = 8
    rows_per_w = R // NW
    n_chunks = rows_per_w // CH
    n_groups = n_chunks // 2
    mesh = plsc.VectorSubcoreMesh(core_axis_name="c", subcore_axis_name="s")

    @functools.partial(
        pl.kernel,
        out_type=jax.ShapeDtypeStruct((R, D), jnp.float32),
        mesh=mesh,
        scratch_types=[
            pltpu.VMEM((CH, D), jnp.float32),
            pltpu.VMEM((CH, D), jnp.float32),
            pltpu.VMEM((CH, D), jnp.float32),
            pltpu.VMEM((CH, D), jnp.float32),
            pltpu.SemaphoreType.DMA,
            pltpu.SemaphoreType.DMA,
            pltpu.SemaphoreType.DMA,
            pltpu.SemaphoreType.DMA,
            pltpu.SemaphoreType.DMA,
            pltpu.SemaphoreType.DMA,
        ],
    )
    def sc_add(x_hbm, e_hbm, out_hbm, bx0, be0, bx1, be1, lx0, le0, lx1,
               le1, so0, so1):
        bufx = (bx0, bx1)
        bufe = (be0, be1)
        slx = (lx0, lx1)
        sle = (le0, le1)
        sso = (so0, so1)
        wid = lax.axis_index("s") * NC + lax.axis_index("c")
        w_base = wid * rows_per_w

        def rows_at(k):
            return pl.ds(w_base + k * CH, CH)

        for b in range(2):
            pltpu.async_copy(x_hbm.at[rows_at(b)], bufx[b], slx[b])
            pltpu.async_copy(e_hbm.at[rows_at(b)], bufe[b], sle[b])

        def group_body(g, carry):
            for b in range(2):
                k = g * 2 + b
                rows = rows_at(k)
                pltpu.make_async_copy(
                    x_hbm.at[rows], bufx[b], slx[b]
                ).wait()
                pltpu.make_async_copy(
                    e_hbm.at[rows], bufe[b], sle[b]
                ).wait()

                for r in range(CH):

                    @plsc.parallel_loop(0, D, 16, unroll=8)
                    def _body(i):
                        plsc.addupdate(
                            bufx[b].at[r, pl.ds(i, 16)],
                            bufe[b][r, pl.ds(i, 16)],
                        )

                pltpu.async_copy(bufx[b], out_hbm.at[rows], sso[b])

                @pl.when(k + 2 < n_chunks)
                def _next():
                    pltpu.make_async_copy(
                        bufx[b], out_hbm.at[rows], sso[b]
                    ).wait()
                    nxt = rows_at(k + 2)
                    pltpu.async_copy(x_hbm.at[nxt], bufx[b], slx[b])
                    pltpu.async_copy(e_hbm.at[nxt], bufe[b], sle[b])

            return carry

        lax.fori_loop(0, n_groups, group_body, 0)

        for b in range(2):
            pltpu.make_async_copy(
                bufx[b], out_hbm.at[rows_at(n_chunks - 2 + b)], sso[b]
            ).wait()

    return sc_add


def kernel(x, node_pos_emb):
    B, L, D = x.shape
    R = B * L
    x2 = x.reshape(R, D)
    e2 = node_pos_emb.reshape(R, D)
    out = _make_sc_add(R, D)(x2, e2)
    return out.reshape(B, L, D)
